# Initial kernel scaffold; baseline (speedup 1.0000x reference)
import jax
import jax.numpy as jnp
from jax.experimental import pallas as pl


def kernel(loc_data, conf_data, priors, mask1, mask2, mask3, targets):
    raise NotImplementedError("write your pallas kernel here")


# speedup vs baseline: 9.5180x; 9.5180x over previous
import jax
import jax.numpy as jnp
from jax import lax
from jax.experimental import pallas as pl
from jax.experimental.pallas import tpu as pltpu

_B = 16
_P = 8732
_C = 21
_O = 5
_R = 69
_L = 128
_PP = _R * _L
_THRESHOLD = 0.5
_NEGPOS = 3
_V0 = 0.1
_V1 = 0.2
_SCALES = (50, 25, 13)
_NPIX = _B * sum(s * s for s in _SCALES)


def _mask_kernel(tg_ref, m1_ref, m2_ref, m3_ref, acc_ref, out_ref):
    b = pl.program_id(0)

    @pl.when(b == 0)
    def _():
        acc_ref[0, 0] = 0.0

    part = 0.0
    for m_ref, S in ((m1_ref, 50), (m2_ref, 25), (m3_ref, 13)):
        x = m_ref[0]
        ys = lax.broadcasted_iota(jnp.int32, (S, S), 0).astype(jnp.float32)
        xs = lax.broadcasted_iota(jnp.int32, (S, S), 1).astype(jnp.float32)
        label = jnp.zeros((S, S), jnp.float32)
        for j in range(_O):
            tx0 = tg_ref[b, j, 0]
            ty0 = tg_ref[b, j, 1]
            tx1 = tg_ref[b, j, 2]
            ty1 = tg_ref[b, j, 3]
            tl = tg_ref[b, j, 4]
            xmin = jnp.maximum(jnp.floor(S * tx0), 0.0)
            ymin = jnp.maximum(jnp.floor(S * ty0), 0.0)
            xmax = jnp.minimum(jnp.ceil(S * tx1 + 1.0), float(S))
            ymax = jnp.minimum(jnp.ceil(S * ty1 + 1.0), float(S))
            cond = ((ys >= ymin) & (ys < ymax) & (xs >= xmin) & (xs < xmax))
            label = jnp.where(cond, tl + 1.0, label)
        m = jnp.max(x, axis=0)
        s = jnp.sum(jnp.exp(x - m[None]), axis=0)
        lse = jnp.log(s) + m
        ci = lax.broadcasted_iota(jnp.int32, (_C, S, S), 0).astype(jnp.float32)
        picked = jnp.sum(jnp.where(ci == label[None], x, 0.0), axis=0)
        part = part + jnp.sum(lse - picked)
    acc_ref[0, 0] += part

    @pl.when(b == _B - 1)
    def _():
        out_ref[0, 0] = acc_ref[0, 0] / float(_NPIX)


def _match_kernel(tg_ref, pri_ref, loc_ref, conf_ref,
                  ce_ref, posf_ref, lossl_ref, n_ref):
    b = pl.program_id(0)

    @pl.when(b == 0)
    def _():
        lossl_ref[0, 0] = 0.0
        n_ref[0, 0] = 0.0

    cx = pri_ref[0]
    cy = pri_ref[1]
    w = pri_ref[2]
    h = pri_ref[3]
    pxmin = cx - w * 0.5
    pymin = cy - h * 0.5
    pxmax = cx + w * 0.5
    pymax = cy + h * 0.5
    area_p = w * h
    pidx = (lax.broadcasted_iota(jnp.int32, (_R, _L), 0) * _L
            + lax.broadcasted_iota(jnp.int32, (_R, _L), 1))

    bto = jnp.full((_R, _L), -1.0, jnp.float32)
    bti = jnp.zeros((_R, _L), jnp.int32)
    bp = []
    for j in range(_O):
        tx0 = tg_ref[b, j, 0]
        ty0 = tg_ref[b, j, 1]
        tx1 = tg_ref[b, j, 2]
        ty1 = tg_ref[b, j, 3]
        ix = jnp.maximum(jnp.minimum(tx1, pxmax) - jnp.maximum(tx0, pxmin), 0.0)
        iy = jnp.maximum(jnp.minimum(ty1, pymax) - jnp.maximum(ty0, pymin), 0.0)
        inter = ix * iy
        area_t = (tx1 - tx0) * (ty1 - ty0)
        iou = inter / (area_t + area_p - inter)
        upd = iou > bto
        bti = jnp.where(upd, j, bti)
        bto = jnp.where(upd, iou, bto)
        mj = jnp.max(iou)
        bpj = jnp.min(jnp.where(iou == mj, pidx, _PP))
        bp.append(bpj)
    for j in range(_O):
        cond = pidx == bp[j]
        bto = jnp.where(cond, 2.0, bto)
        bti = jnp.where(cond, j, bti)

    lab = jnp.zeros((_R, _L), jnp.float32)
    mt0 = jnp.zeros((_R, _L), jnp.float32)
    mt1 = jnp.zeros((_R, _L), jnp.float32)
    mt2 = jnp.zeros((_R, _L), jnp.float32)
    mt3 = jnp.zeros((_R, _L), jnp.float32)
    for j in range(_O):
        sel = bti == j
        lab = jnp.where(sel, tg_ref[b, j, 4], lab)
        mt0 = jnp.where(sel, tg_ref[b, j, 0], mt0)
        mt1 = jnp.where(sel, tg_ref[b, j, 1], mt1)
        mt2 = jnp.where(sel, tg_ref[b, j, 2], mt2)
        mt3 = jnp.where(sel, tg_ref[b, j, 3], mt3)
    valid = pidx < _P
    conf_t = jnp.where((bto < _THRESHOLD) | ~valid, 0.0, lab + 1.0)
    posb = conf_t > 0.0
    posf = jnp.where(posb, 1.0, 0.0)

    g0 = ((mt0 + mt2) * 0.5 - cx) / (_V0 * w)
    g1 = ((mt1 + mt3) * 0.5 - cy) / (_V0 * h)
    g2 = jnp.log(jnp.maximum(mt2 - mt0, 1e-10) / w) / _V1
    g3 = jnp.log(jnp.maximum(mt3 - mt1, 1e-10) / h) / _V1
    sl = 0.0
    for k, g in ((0, g0), (1, g1), (2, g2), (3, g3)):
        d = loc_ref[0, k] - g
        ad = jnp.abs(d)
        sl = sl + jnp.where(ad < 1.0, 0.5 * d * d, ad - 0.5)
    lossl_ref[0, 0] += jnp.sum(jnp.where(posb, sl, 0.0))
    n_ref[0, 0] += jnp.sum(posf)

    x = conf_ref[0]
    m = jnp.max(x, axis=0)
    s = jnp.sum(jnp.exp(x - m[None]), axis=0)
    lse = jnp.log(s) + m
    ci = lax.broadcasted_iota(jnp.int32, (_C, _R, _L), 0).astype(jnp.float32)
    picked = jnp.sum(jnp.where(ci == conf_t[None], x, 0.0), axis=0)
    ce = jnp.where(valid, lse - picked, 0.0)
    ce_ref[0] = ce
    posf_ref[0] = posf


def _mine_kernel(ce_ref, posf_ref, lossl_ref, n_ref, ll_ref, lc_ref):
    ce = ce_ref[...]
    posf = posf_ref[...]
    posb = posf > 0.0
    v = jnp.where(posb, 0.0, ce)
    bits = lax.bitcast_convert_type(v, jnp.int32)
    pidx = (lax.broadcasted_iota(jnp.int32, (_B, _R, _L), 1) * _L
            + lax.broadcasted_iota(jnp.int32, (_B, _R, _L), 2))
    num_pos = jnp.sum(posf, axis=(1, 2), keepdims=True)
    k = jnp.minimum(_NEGPOS * num_pos, float(_P - 1))

    def srch(i, carry):
        lo, hi = carry
        mid = lo + (hi - lo + 1) // 2
        cnt = jnp.sum(jnp.where(bits >= mid, 1.0, 0.0), axis=(1, 2),
                      keepdims=True)
        ok = cnt >= k
        return jnp.where(ok, mid, lo), jnp.where(ok, hi, mid - 1)

    lo0 = jnp.zeros((_B, 1, 1), jnp.int32)
    hi0 = jnp.full((_B, 1, 1), 0x7f800000, jnp.int32)
    t, _unused = lax.fori_loop(0, 31, srch, (lo0, hi0))

    gt = bits > t
    eq = bits == t
    need = k - jnp.sum(jnp.where(gt, 1.0, 0.0), axis=(1, 2), keepdims=True)

    def srch2(i, carry):
        lo, hi = carry
        mid = (lo + hi) // 2
        cnt = jnp.sum(jnp.where(eq & (pidx <= mid), 1.0, 0.0), axis=(1, 2),
                      keepdims=True)
        ok = cnt >= need
        return jnp.where(ok, lo, mid + 1), jnp.where(ok, mid, hi)

    lo0 = jnp.zeros((_B, 1, 1), jnp.int32)
    hi0 = jnp.full((_B, 1, 1), _PP - 1, jnp.int32)
    idx_t, _unused2 = lax.fori_loop(0, 14, srch2, (lo0, hi0))

    sel = posb | gt | (eq & (pidx <= idx_t))
    csum = jnp.sum(jnp.where(sel, ce, 0.0))
    n = n_ref[0, 0]
    ll_ref[0, 0] = lossl_ref[0, 0] / n * 2.0
    lc_ref[0, 0] = csum / n * 2.0


def kernel(loc_data, conf_data, priors, mask1, mask2, mask3, targets):
    conf_t = jnp.transpose(conf_data, (0, 2, 1))
    conf_t = jnp.pad(conf_t, ((0, 0), (0, 0), (0, _PP - _P)))
    conf_t = conf_t.reshape(_B, _C, _R, _L)
    loc_t = jnp.transpose(loc_data, (0, 2, 1))
    loc_t = jnp.pad(loc_t, ((0, 0), (0, 0), (0, _PP - _P)))
    loc_t = loc_t.reshape(_B, 4, _R, _L)
    pri_t = jnp.transpose(priors, (1, 0))
    pad_vals = jnp.tile(jnp.array([[3.0], [3.0], [1.0], [1.0]], jnp.float32),
                        (1, _PP - _P))
    pri_t = jnp.concatenate([pri_t, pad_vals], axis=1).reshape(4, _R, _L)

    smem = pl.BlockSpec(memory_space=pltpu.SMEM)

    loss_m = pl.pallas_call(
        _mask_kernel,
        grid=(_B,),
        in_specs=[
            smem,
            pl.BlockSpec((1, _C, 50, 50), lambda b: (b, 0, 0, 0)),
            pl.BlockSpec((1, _C, 25, 25), lambda b: (b, 0, 0, 0)),
            pl.BlockSpec((1, _C, 13, 13), lambda b: (b, 0, 0, 0)),
        ],
        out_specs=[
            pl.BlockSpec(memory_space=pltpu.SMEM),
            pl.BlockSpec(memory_space=pltpu.SMEM),
        ],
        out_shape=[
            jax.ShapeDtypeStruct((1, 1), jnp.float32),
            jax.ShapeDtypeStruct((1, 1), jnp.float32),
        ],
    )(targets, mask1, mask2, mask3)[1]

    ce, posf, lossl, n = pl.pallas_call(
        _match_kernel,
        grid=(_B,),
        in_specs=[
            smem,
            pl.BlockSpec((4, _R, _L), lambda b: (0, 0, 0)),
            pl.BlockSpec((1, 4, _R, _L), lambda b: (b, 0, 0, 0)),
            pl.BlockSpec((1, _C, _R, _L), lambda b: (b, 0, 0, 0)),
        ],
        out_specs=[
            pl.BlockSpec((1, _R, _L), lambda b: (b, 0, 0)),
            pl.BlockSpec((1, _R, _L), lambda b: (b, 0, 0)),
            pl.BlockSpec(memory_space=pltpu.SMEM),
            pl.BlockSpec(memory_space=pltpu.SMEM),
        ],
        out_shape=[
            jax.ShapeDtypeStruct((_B, _R, _L), jnp.float32),
            jax.ShapeDtypeStruct((_B, _R, _L), jnp.float32),
            jax.ShapeDtypeStruct((1, 1), jnp.float32),
            jax.ShapeDtypeStruct((1, 1), jnp.float32),
        ],
    )(targets, pri_t, loc_t, conf_t)

    ll, lc = pl.pallas_call(
        _mine_kernel,
        in_specs=[
            pl.BlockSpec((_B, _R, _L), lambda: (0, 0, 0)),
            pl.BlockSpec((_B, _R, _L), lambda: (0, 0, 0)),
            smem,
            smem,
        ],
        out_specs=[
            pl.BlockSpec(memory_space=pltpu.SMEM),
            pl.BlockSpec(memory_space=pltpu.SMEM),
        ],
        out_shape=[
            jax.ShapeDtypeStruct((1, 1), jnp.float32),
            jax.ShapeDtypeStruct((1, 1), jnp.float32),
        ],
    )(ce, posf, lossl, n)

    return ll[0, 0], lc[0, 0], loss_m[0, 0]


# speedup vs baseline: 9.6055x; 1.0092x over previous
import jax
import jax.numpy as jnp
from jax import lax
from jax.experimental import pallas as pl
from jax.experimental.pallas import tpu as pltpu

_B = 16
_P = 8732
_C = 21
_O = 5
_R = 69
_L = 128
_PP = _R * _L
_THRESHOLD = 0.5
_NEGPOS = 3
_V0 = 0.1
_V1 = 0.2
_SCALES = (50, 25, 13)
_NPIX = _B * sum(s * s for s in _SCALES)


def _fused_kernel(tg_ref,
                  ys1_ref, xs1_ref, m1_ref,
                  ys2_ref, xs2_ref, m2_ref,
                  ys3_ref, xs3_ref, m3_ref,
                  pri_ref, loc_ref, conf_ref,
                  ll_ref, lc_ref, lm_ref,
                  v_scr, np_scr, acc_ref):
    b = pl.program_id(0)

    @pl.when(b == 0)
    def _():
        acc_ref[0] = 0.0
        acc_ref[1] = 0.0
        acc_ref[2] = 0.0
        acc_ref[3] = 0.0

    mpart = 0.0
    for ys_ref, xs_ref, m_ref, S in ((ys1_ref, xs1_ref, m1_ref, 50),
                                     (ys2_ref, xs2_ref, m2_ref, 25),
                                     (ys3_ref, xs3_ref, m3_ref, 13)):
        S2 = S * S
        x = m_ref[0]
        ys = ys_ref[...]
        xs = xs_ref[...]
        label = jnp.zeros((1, S2), jnp.float32)
        for j in range(_O):
            tx0 = tg_ref[b, j, 0]
            ty0 = tg_ref[b, j, 1]
            tx1 = tg_ref[b, j, 2]
            ty1 = tg_ref[b, j, 3]
            tl = tg_ref[b, j, 4]
            xmin = jnp.maximum(jnp.floor(S * tx0), 0.0)
            ymin = jnp.maximum(jnp.floor(S * ty0), 0.0)
            xmax = jnp.minimum(jnp.ceil(S * tx1 + 1.0), float(S))
            ymax = jnp.minimum(jnp.ceil(S * ty1 + 1.0), float(S))
            cond = ((ys >= ymin) & (ys < ymax) & (xs >= xmin) & (xs < xmax))
            label = jnp.where(cond, tl + 1.0, label)
        m = jnp.max(x, axis=0, keepdims=True)
        s = jnp.sum(jnp.exp(x - m), axis=0, keepdims=True)
        lse = jnp.log(s) + m
        ci = lax.broadcasted_iota(jnp.int32, (_C, S2), 0).astype(jnp.float32)
        picked = jnp.sum(jnp.where(ci == label, x, 0.0), axis=0, keepdims=True)
        mpart = mpart + jnp.sum(lse - picked)
    acc_ref[0] += mpart

    cx = pri_ref[0]
    cy = pri_ref[1]
    w = pri_ref[2]
    h = pri_ref[3]
    pxmin = cx - w * 0.5
    pymin = cy - h * 0.5
    pxmax = cx + w * 0.5
    pymax = cy + h * 0.5
    area_p = w * h
    pidx = (lax.broadcasted_iota(jnp.int32, (_R, _L), 0) * _L
            + lax.broadcasted_iota(jnp.int32, (_R, _L), 1))

    bto = jnp.full((_R, _L), -1.0, jnp.float32)
    bti = jnp.zeros((_R, _L), jnp.int32)
    bp = []
    for j in range(_O):
        tx0 = tg_ref[b, j, 0]
        ty0 = tg_ref[b, j, 1]
        tx1 = tg_ref[b, j, 2]
        ty1 = tg_ref[b, j, 3]
        ix = jnp.maximum(jnp.minimum(tx1, pxmax) - jnp.maximum(tx0, pxmin), 0.0)
        iy = jnp.maximum(jnp.minimum(ty1, pymax) - jnp.maximum(ty0, pymin), 0.0)
        inter = ix * iy
        area_t = (tx1 - tx0) * (ty1 - ty0)
        iou = inter / (area_t + area_p - inter)
        upd = iou > bto
        bti = jnp.where(upd, j, bti)
        bto = jnp.where(upd, iou, bto)
        mj = jnp.max(iou)
        bpj = jnp.min(jnp.where(iou == mj, pidx, _PP))
        bp.append(bpj)
    for j in range(_O):
        cond = pidx == bp[j]
        bto = jnp.where(cond, 2.0, bto)
        bti = jnp.where(cond, j, bti)

    lab = jnp.zeros((_R, _L), jnp.float32)
    mt0 = jnp.zeros((_R, _L), jnp.float32)
    mt1 = jnp.zeros((_R, _L), jnp.float32)
    mt2 = jnp.zeros((_R, _L), jnp.float32)
    mt3 = jnp.zeros((_R, _L), jnp.float32)
    for j in range(_O):
        sel = bti == j
        lab = jnp.where(sel, tg_ref[b, j, 4], lab)
        mt0 = jnp.where(sel, tg_ref[b, j, 0], mt0)
        mt1 = jnp.where(sel, tg_ref[b, j, 1], mt1)
        mt2 = jnp.where(sel, tg_ref[b, j, 2], mt2)
        mt3 = jnp.where(sel, tg_ref[b, j, 3], mt3)
    valid = pidx < _P
    conf_t = jnp.where((bto < _THRESHOLD) | ~valid, 0.0, lab + 1.0)
    posb = conf_t > 0.0
    posf = jnp.where(posb, 1.0, 0.0)

    g0 = ((mt0 + mt2) * 0.5 - cx) / (_V0 * w)
    g1 = ((mt1 + mt3) * 0.5 - cy) / (_V0 * h)
    g2 = jnp.log(jnp.maximum(mt2 - mt0, 1e-10) / w) / _V1
    g3 = jnp.log(jnp.maximum(mt3 - mt1, 1e-10) / h) / _V1
    sl = 0.0
    for k, g in ((0, g0), (1, g1), (2, g2), (3, g3)):
        d = loc_ref[0, k] - g
        ad = jnp.abs(d)
        sl = sl + jnp.where(ad < 1.0, 0.5 * d * d, ad - 0.5)
    acc_ref[1] += jnp.sum(jnp.where(posb, sl, 0.0))
    npsum = jnp.sum(posf)
    acc_ref[2] += npsum

    x = conf_ref[0]
    m = jnp.max(x, axis=0)
    s = jnp.sum(jnp.exp(x - m[None]), axis=0)
    lse = jnp.log(s) + m
    ci = lax.broadcasted_iota(jnp.int32, (_C, _R, _L), 0).astype(jnp.float32)
    picked = jnp.sum(jnp.where(ci == conf_t[None], x, 0.0), axis=0)
    ce = jnp.where(valid, lse - picked, 0.0)
    acc_ref[3] += jnp.sum(jnp.where(posb, ce, 0.0))
    v_scr[b] = jnp.where(posb, 0.0, ce)
    np_scr[b] = jnp.zeros((_L,), jnp.float32) + npsum

    @pl.when(b == _B - 1)
    def _():
        v = v_scr[...]
        bits = lax.bitcast_convert_type(v, jnp.int32)
        pidx3 = (lax.broadcasted_iota(jnp.int32, (_B, _R, _L), 1) * _L
                 + lax.broadcasted_iota(jnp.int32, (_B, _R, _L), 2))
        num_pos = np_scr[...][:, 0:1].reshape(_B, 1, 1)
        kk = jnp.minimum(_NEGPOS * num_pos, float(_P - 1))

        def srch(i, carry):
            lo, hi = carry
            mid = lo + (hi - lo + 1) // 2
            cnt = jnp.sum(jnp.where(bits >= mid, 1.0, 0.0), axis=(1, 2),
                          keepdims=True)
            ok = cnt >= kk
            return jnp.where(ok, mid, lo), jnp.where(ok, hi, mid - 1)

        lo0 = jnp.zeros((_B, 1, 1), jnp.int32)
        hi0 = jnp.full((_B, 1, 1), 0x7f800000, jnp.int32)
        t, _u1 = lax.fori_loop(0, 31, srch, (lo0, hi0))

        gt = bits > t
        eq = bits == t
        need = kk - jnp.sum(jnp.where(gt, 1.0, 0.0), axis=(1, 2),
                            keepdims=True)

        def srch2(i, carry):
            lo, hi = carry
            mid = (lo + hi) // 2
            cnt = jnp.sum(jnp.where(eq & (pidx3 <= mid), 1.0, 0.0),
                          axis=(1, 2), keepdims=True)
            ok = cnt >= need
            return jnp.where(ok, lo, mid + 1), jnp.where(ok, mid, hi)

        lo0 = jnp.zeros((_B, 1, 1), jnp.int32)
        hi0 = jnp.full((_B, 1, 1), _PP - 1, jnp.int32)
        idx_t, _u2 = lax.fori_loop(0, 14, srch2, (lo0, hi0))

        neg = gt | (eq & (pidx3 <= idx_t))
        negsum = jnp.sum(jnp.where(neg, v, 0.0))
        n = acc_ref[2]
        ll_ref[0, 0] = acc_ref[1] / n * 2.0
        lc_ref[0, 0] = (acc_ref[3] + negsum) / n * 2.0
        lm_ref[0, 0] = acc_ref[0] / float(_NPIX)


def _coords(S):
    p = jnp.arange(S * S, dtype=jnp.int32)
    return ((p // S).astype(jnp.float32).reshape(1, S * S),
            (p % S).astype(jnp.float32).reshape(1, S * S))


def kernel(loc_data, conf_data, priors, mask1, mask2, mask3, targets):
    conf_t = jnp.transpose(conf_data, (0, 2, 1))
    conf_t = jnp.pad(conf_t, ((0, 0), (0, 0), (0, _PP - _P)))
    conf_t = conf_t.reshape(_B, _C, _R, _L)
    loc_t = jnp.transpose(loc_data, (0, 2, 1))
    loc_t = jnp.pad(loc_t, ((0, 0), (0, 0), (0, _PP - _P)))
    loc_t = loc_t.reshape(_B, 4, _R, _L)
    pri_t = jnp.transpose(priors, (1, 0))
    pad_vals = jnp.tile(jnp.array([[3.0], [3.0], [1.0], [1.0]], jnp.float32),
                        (1, _PP - _P))
    pri_t = jnp.concatenate([pri_t, pad_vals], axis=1).reshape(4, _R, _L)
    m1 = mask1.reshape(_B, _C, 2500)
    m2 = mask2.reshape(_B, _C, 625)
    m3 = mask3.reshape(_B, _C, 169)
    ys1, xs1 = _coords(50)
    ys2, xs2 = _coords(25)
    ys3, xs3 = _coords(13)

    smem = pl.BlockSpec(memory_space=pltpu.SMEM)

    def cspec(shape):
        return pl.BlockSpec(shape, lambda b: tuple(0 for _ in shape))

    ll, lc, lm = pl.pallas_call(
        _fused_kernel,
        grid=(_B,),
        in_specs=[
            smem,
            cspec((1, 2500)), cspec((1, 2500)),
            pl.BlockSpec((1, _C, 2500), lambda b: (b, 0, 0)),
            cspec((1, 625)), cspec((1, 625)),
            pl.BlockSpec((1, _C, 625), lambda b: (b, 0, 0)),
            cspec((1, 169)), cspec((1, 169)),
            pl.BlockSpec((1, _C, 169), lambda b: (b, 0, 0)),
            cspec((4, _R, _L)),
            pl.BlockSpec((1, 4, _R, _L), lambda b: (b, 0, 0, 0)),
            pl.BlockSpec((1, _C, _R, _L), lambda b: (b, 0, 0, 0)),
        ],
        out_specs=[
            pl.BlockSpec(memory_space=pltpu.SMEM),
            pl.BlockSpec(memory_space=pltpu.SMEM),
            pl.BlockSpec(memory_space=pltpu.SMEM),
        ],
        out_shape=[
            jax.ShapeDtypeStruct((1, 1), jnp.float32),
            jax.ShapeDtypeStruct((1, 1), jnp.float32),
            jax.ShapeDtypeStruct((1, 1), jnp.float32),
        ],
        scratch_shapes=[
            pltpu.VMEM((_B, _R, _L), jnp.float32),
            pltpu.VMEM((_B, _L), jnp.float32),
            pltpu.SMEM((4,), jnp.float32),
        ],
    )(targets, ys1, xs1, m1, ys2, xs2, m2, ys3, xs3, m3, pri_t, loc_t, conf_t)

    return ll[0, 0], lc[0, 0], lm[0, 0]
